# Initial kernel scaffold; baseline (speedup 1.0000x reference)
import jax
import jax.numpy as jnp
from jax.experimental import pallas as pl


def kernel(x, edge_index, edge_offset, W, gamma, beta):
    raise NotImplementedError("write your pallas kernel here")


# speedup vs baseline: 3.0712x; 3.0712x over previous
import functools

import jax
import jax.numpy as jnp
from jax import lax
from jax.experimental import pallas as pl
from jax.experimental.pallas import tpu as pltpu
from jax.experimental.pallas import tpu_sc as plsc

N = 10000
E = 320000
C_IN = 128
C_OUT = 128
KVOL = 27
EPS = 1e-5

NUM_CORES = 2
NUM_SUBCORES = 16
NUM_WORKERS = NUM_CORES * NUM_SUBCORES

CHUNK = 80
ROWS_PER_W = E // NUM_WORKERS // CHUNK
NPAD = 10240
TPN = NPAD // NUM_SUBCORES

NB = 2000
NBC = N // NB


def _h_body(x_ref, w_ref, h_ref):
    h_ref[0] = jnp.dot(x_ref[...], w_ref[0], preferred_element_type=jnp.float32)


def _gidx_body(o_ref, s_ref, g_ref):
    g_ref[...] = o_ref[...] * N + s_ref[...]


def _bn_body(p_ref, g_ref, b_ref, o_ref):
    s = p_ref[0] + p_ref[1]
    m = jnp.mean(s, axis=0, keepdims=True)
    v = jnp.mean((s - m) ** 2, axis=0, keepdims=True)
    o_ref[...] = jnp.maximum((s - m) * lax.rsqrt(v + EPS) * g_ref[...] + b_ref[...],
                             0.0)


def _sc_body(h_hbm, gidx_hbm, dst_hbm, zero_hbm, out_hbm,
             gidx_v, dst_v, rows_v, acc, sem):
    cid = lax.axis_index("c")
    sid = lax.axis_index("s")
    w = sid * NUM_CORES + cid

    pltpu.sync_copy(zero_hbm.at[pl.ds(sid * TPN, TPN)],
                    acc.at[pl.ds(sid * TPN, TPN)])

    pltpu.sync_copy(gidx_hbm.at[w], gidx_v)
    pltpu.sync_copy(dst_hbm.at[w], dst_v)
    plsc.subcore_barrier()

    def body(i, carry):
        pltpu.async_copy(h_hbm.at[gidx_v.at[i]], rows_v, sem).wait()
        pltpu.sync_copy(rows_v, acc.at[dst_v.at[i]], add=True)
        return carry

    lax.fori_loop(0, ROWS_PER_W, body, 0)
    plsc.subcore_barrier()

    pltpu.sync_copy(acc.at[pl.ds(sid * TPN, TPN)],
                    out_hbm.at[cid, pl.ds(sid * TPN, TPN)])


@functools.cache
def _sc_scatter():
    return pl.kernel(
        _sc_body,
        out_type=jax.ShapeDtypeStruct((NUM_CORES, NPAD, C_OUT), jnp.float32),
        mesh=plsc.VectorSubcoreMesh(core_axis_name="c", subcore_axis_name="s",
                                    num_cores=NUM_CORES,
                                    num_subcores=NUM_SUBCORES),
        scratch_types=[
            pltpu.VMEM((ROWS_PER_W, CHUNK), jnp.int32),
            pltpu.VMEM((ROWS_PER_W, CHUNK), jnp.int32),
            pltpu.VMEM((CHUNK, C_OUT), jnp.float32),
            pltpu.VMEM_SHARED((NPAD, C_OUT), jnp.float32),
            pltpu.SemaphoreType.DMA,
        ],
    )


def kernel(x, edge_index, edge_offset, W, gamma, beta):
    src = edge_index[0]
    dst = edge_index[1]

    H = pl.pallas_call(
        _h_body,
        grid=(NBC, KVOL),
        in_specs=[
            pl.BlockSpec((NB, C_IN), lambda nb, k: (nb, 0)),
            pl.BlockSpec((1, C_IN, C_OUT), lambda nb, k: (k, 0, 0)),
        ],
        out_specs=pl.BlockSpec((1, NB, C_OUT), lambda nb, k: (k, nb, 0)),
        out_shape=jax.ShapeDtypeStruct((KVOL, N, C_OUT), jnp.float32),
    )(x, W)
    H2 = H.reshape(KVOL * N, C_OUT)

    gidx = pl.pallas_call(
        _gidx_body,
        out_shape=jax.ShapeDtypeStruct((E // C_OUT, C_OUT), jnp.int32),
    )(edge_offset.reshape(E // C_OUT, C_OUT), src.reshape(E // C_OUT, C_OUT))

    zeros = jnp.zeros((NPAD, C_OUT), jnp.float32)
    parts = _sc_scatter()(
        H2,
        gidx.reshape(NUM_WORKERS, ROWS_PER_W, CHUNK),
        dst.reshape(NUM_WORKERS, ROWS_PER_W, CHUNK),
        zeros,
    )

    out = pl.pallas_call(
        _bn_body,
        out_shape=jax.ShapeDtypeStruct((N, C_OUT), jnp.float32),
    )(parts[:, :N], gamma.reshape(1, C_OUT), beta.reshape(1, C_OUT))
    return out
